# Initial kernel scaffold; baseline (speedup 1.0000x reference)
import jax
import jax.numpy as jnp
from jax.experimental import pallas as pl


def kernel(token_indices, table, pos_enc):
    raise NotImplementedError("write your pallas kernel here")


# speedup vs baseline: 4.0551x; 4.0551x over previous
import functools

import jax
import jax.numpy as jnp
from jax import lax
from jax.experimental import pallas as pl
from jax.experimental.pallas import tpu as pltpu
from jax.experimental.pallas import tpu_sc as plsc

B, L, D = 4096, 200, 64
VOCAB1 = 69
PAD_ID = 68
N = B * L
IDX_MINOR = 128
IDX_ROWS = N // IDX_MINOR

NC, NS = 2, 16
NW = NC * NS
ROWS_PER_W = N // NW
IDXR_PER_W = IDX_ROWS // NW
G_ROWS = 4
NG = IDXR_PER_W // G_ROWS
G_OUT = G_ROWS * IDX_MINOR


def _prep_body(tok_ref, table_ref, pos_ref, comb_ref, idx_ref, mask_ref):
    tok = tok_ref[...]
    l_pos = lax.broadcasted_iota(jnp.int32, (B, L), 1)
    idx_ref[...] = tok + VOCAB1 * l_pos
    mask_ref[...] = tok == PAD_ID
    comb_ref[...] = pos_ref[0:L, :][:, None, :] + table_ref[...][None, :, :]


_prep = pl.pallas_call(
    _prep_body,
    out_shape=[
        jax.ShapeDtypeStruct((L, VOCAB1, D), jnp.float32),
        jax.ShapeDtypeStruct((B, L), jnp.int32),
        jax.ShapeDtypeStruct((B, L), jnp.bool_),
    ],
)


@functools.cache
def _make_sc_gather():
    @functools.partial(
        pl.kernel,
        out_type=jax.ShapeDtypeStruct((N, D), jnp.float32),
        mesh=plsc.VectorSubcoreMesh(
            core_axis_name="c", subcore_axis_name="s",
            num_cores=NC, num_subcores=NS,
        ),
        scratch_types=[
            pltpu.VMEM((G_ROWS, IDX_MINOR), jnp.int32),
            pltpu.VMEM((G_OUT, D), jnp.float32),
            pltpu.SemaphoreType.DMA,
        ],
        compiler_params=pltpu.CompilerParams(use_tc_tiling_on_sc=False),
    )
    def _sc_gather(comb_hbm, idx_hbm, out_hbm, idx_v, out_v, sem):
        wid = lax.axis_index("s") * NC + lax.axis_index("c")

        def group(g, carry):
            r0 = wid * IDXR_PER_W + g * G_ROWS
            pltpu.sync_copy(idx_hbm.at[pl.ds(r0, G_ROWS)], idx_v)
            descs = [
                pltpu.async_copy(
                    comb_hbm.at[idx_v.at[j]],
                    out_v.at[pl.ds(j * IDX_MINOR, IDX_MINOR)],
                    sem,
                )
                for j in range(G_ROWS)
            ]
            for d in descs:
                d.wait()
            pltpu.sync_copy(
                out_v, out_hbm.at[pl.ds(wid * ROWS_PER_W + g * G_OUT, G_OUT)]
            )
            return carry

        lax.fori_loop(0, NG, group, 0)

    return _sc_gather


def kernel(token_indices, table, pos_enc):
    comb3, flat_idx, mask = _prep(token_indices, table, pos_enc)
    comb = comb3.reshape(L * VOCAB1, D)
    idx = flat_idx.reshape(IDX_ROWS, IDX_MINOR)
    out = _make_sc_gather()(comb, idx)
    return (out.reshape(B, L, D), token_indices, mask)


# speedup vs baseline: 4.2659x; 1.0520x over previous
import functools

import jax
import jax.numpy as jnp
from jax import lax
from jax.experimental import pallas as pl
from jax.experimental.pallas import tpu as pltpu
from jax.experimental.pallas import tpu_sc as plsc

B, L, D = 4096, 200, 64
VOCAB1 = 69
PAD_ID = 68
N = B * L
IDX_MINOR = 128
IDX_ROWS = N // IDX_MINOR

NC, NS = 2, 16
NW = NC * NS
ROWS_PER_W = N // NW
IDXR_PER_W = IDX_ROWS // NW
G_ROWS = 4
NG = IDXR_PER_W // G_ROWS
G_OUT = G_ROWS * IDX_MINOR


def _prep_body(tok_ref, table_ref, pos_ref, comb_ref, idx_ref, mask_ref):
    tok = tok_ref[...]
    l_pos = lax.broadcasted_iota(jnp.int32, (B, L), 1)
    idx_ref[...] = tok + VOCAB1 * l_pos
    mask_ref[...] = tok == PAD_ID
    comb_ref[...] = pos_ref[0:L, :][:, None, :] + table_ref[...][None, :, :]


_prep = pl.pallas_call(
    _prep_body,
    out_shape=[
        jax.ShapeDtypeStruct((L, VOCAB1, D), jnp.float32),
        jax.ShapeDtypeStruct((B, L), jnp.int32),
        jax.ShapeDtypeStruct((B, L), jnp.bool_),
    ],
)


@functools.cache
def _make_sc_gather():
    @functools.partial(
        pl.kernel,
        out_type=jax.ShapeDtypeStruct((N, D), jnp.float32),
        mesh=plsc.VectorSubcoreMesh(
            core_axis_name="c", subcore_axis_name="s",
            num_cores=NC, num_subcores=NS,
        ),
        scratch_types=[
            pltpu.VMEM((2, G_ROWS, IDX_MINOR), jnp.int32),
            pltpu.VMEM((2, G_OUT, D), jnp.float32),
            pltpu.SemaphoreType.DMA,
            pltpu.SemaphoreType.DMA,
            pltpu.SemaphoreType.DMA,
            pltpu.SemaphoreType.DMA,
        ],
        compiler_params=pltpu.CompilerParams(use_tc_tiling_on_sc=False),
    )
    def _sc_gather(comb_hbm, idx_hbm, out_hbm, idx_v, out_v, g0, g1, s0, s1):
        wid = lax.axis_index("s") * NC + lax.axis_index("c")
        gsem, ssem = (g0, g1), (s0, s1)

        def idx_load(g, b):
            r0 = wid * IDXR_PER_W + g * G_ROWS
            pltpu.sync_copy(idx_hbm.at[pl.ds(r0, G_ROWS)], idx_v.at[b])

        def fire_gathers(g, b):
            for j in range(G_ROWS):
                pltpu.async_copy(
                    comb_hbm.at[idx_v.at[b, j]],
                    out_v.at[b, pl.ds(j * IDX_MINOR, IDX_MINOR)],
                    gsem[b],
                )

        def wait_gathers(b):
            pltpu.make_async_copy(
                comb_hbm.at[pl.ds(0, G_OUT)], out_v.at[b], gsem[b]
            ).wait()

        def fire_scatter(g, b):
            pltpu.async_copy(
                out_v.at[b],
                out_hbm.at[pl.ds(wid * ROWS_PER_W + g * G_OUT, G_OUT)],
                ssem[b],
            )

        def wait_scatter(b):
            pltpu.make_async_copy(
                out_v.at[b], out_hbm.at[pl.ds(0, G_OUT)], ssem[b]
            ).wait()

        idx_load(0, 0)
        fire_gathers(0, 0)

        @pl.loop(0, NG, step=2)
        def _pair(go):
            for half in range(2):
                g = go + half
                b = half
                wait_gathers(b)
                fire_scatter(g, b)

                @pl.when(g + 1 < NG)
                def _prefetch():
                    @pl.when(g >= 1)
                    def _reuse_guard():
                        wait_scatter(1 - b)

                    idx_load(g + 1, 1 - b)
                    fire_gathers(g + 1, 1 - b)

        wait_scatter(0)
        wait_scatter(1)

    return _sc_gather


def kernel(token_indices, table, pos_enc):
    comb3, flat_idx, mask = _prep(token_indices, table, pos_enc)
    comb = comb3.reshape(L * VOCAB1, D)
    idx = flat_idx.reshape(IDX_ROWS, IDX_MINOR)
    out = _make_sc_gather()(comb, idx)
    return (out.reshape(B, L, D), token_indices, mask)
